# baseline (speedup 1.0000x reference)
import functools

import jax
import jax.numpy as jnp
from jax import lax
from jax.experimental import pallas as pl
from jax.experimental.pallas import tpu as pltpu
from jax.experimental.pallas import tpu_sc as plsc

_D = 32
_NC = 2
_NS = 16
_NW = _NC * _NS
_LANES = 16
_BW = 128
_TCH = 10
_NBUF = 2
_REP = 1025


def _lookup(xt, wtab, n_t, n_b):
    chunks = n_t // _TCH
    mesh = plsc.VectorSubcoreMesh(core_axis_name="c", subcore_axis_name="s")

    @functools.partial(
        pl.kernel,
        mesh=mesh,
        compiler_params=pltpu.CompilerParams(needs_layout_passes=False),
        out_type=jax.ShapeDtypeStruct((n_t * _D, n_b), jnp.float32),
        scratch_types=[
            pltpu.VMEM((n_t, _BW), jnp.int32),
            pltpu.VMEM((_D * _D,), jnp.float32),
            pltpu.VMEM((_LANES * _REP,), jnp.float32),
        ] + [pltpu.VMEM((_TCH * _D, _BW), jnp.float32)] * _NBUF
          + [pltpu.SemaphoreType.DMA] * (_NBUF + 1),
    )
    def k(xt_hbm, wtab_hbm, out_hbm, xl_v, wtab_v, rep_v, *bufs_and_sems):
        rows_v = bufs_and_sems[:_NBUF]
        osem = bufs_and_sems[_NBUF:2 * _NBUF]
        xsem = bufs_and_sems[2 * _NBUF]
        wid = lax.axis_index("s") * _NC + lax.axis_index("c")
        b0 = wid * _BW
        iota = lax.iota(jnp.int32, _LANES)
        skew = iota * _REP

        pltpu.async_copy(xt_hbm.at[:, pl.ds(b0, _BW)], xl_v, xsem)
        pltpu.sync_copy(wtab_hbm, wtab_v)

        def rep_body(kk, carry):
            v = wtab_v[pl.ds(kk * _LANES, _LANES)]
            for l in range(_LANES):
                rep_v[pl.ds(l * _REP + kk * _LANES, _LANES)] = v
            return carry

        lax.fori_loop(0, (_D * _D) // _LANES, rep_body, 0)
        pltpu.make_async_copy(xt_hbm.at[:, pl.ds(0, _BW)], xl_v, xsem).wait()

        def pair_body(p, carry):
            for b in range(_NBUF):
                c = p * _NBUF + b
                t0 = c * _TCH
                @pl.when(p > 0)
                def _():
                    pltpu.make_async_copy(
                        rows_v[b], out_hbm.at[pl.ds(0, _TCH * _D),
                                              pl.ds(b0, _BW)],
                        osem[b]).wait()

                @plsc.parallel_loop(0, _TCH * (_BW // _LANES), 1, unroll=2)
                def group_body(i):
                    tl = i // (_BW // _LANES)
                    g = i % (_BW // _LANES)
                    cvec = xl_v[t0 + tl, pl.ds(g * _LANES, _LANES)]
                    coffs = jnp.minimum(jnp.maximum(cvec, 0), _D - 1)
                    bsvec = coffs + skew
                    for d in range(_D):
                        gth = plsc.load_gather(rep_v, [bsvec + d * _D])
                        rows_v[b][tl * _D + d, pl.ds(g * _LANES, _LANES)] = gth

                pltpu.async_copy(
                    rows_v[b],
                    out_hbm.at[pl.ds(t0 * _D, _TCH * _D), pl.ds(b0, _BW)],
                    osem[b])
            return carry

        lax.fori_loop(0, chunks // _NBUF, pair_body, 0)
        for b in range(_NBUF):
            pltpu.make_async_copy(
                rows_v[b], out_hbm.at[pl.ds(0, _TCH * _D), pl.ds(b0, _BW)],
                osem[b]).wait()

    return k(xt, wtab)


def kernel(x, W):
    n_b, n_t = x.shape
    wtab = W[:_D].T.reshape(-1)
    out2 = _lookup(x.T, wtab, n_t, n_b)
    out = out2.reshape(n_t, _D, n_b).transpose(2, 0, 1)
    return out


# speedup vs baseline: 1.0406x; 1.0406x over previous
import functools

import jax
import jax.numpy as jnp
from jax import lax
from jax.experimental import pallas as pl
from jax.experimental.pallas import tpu as pltpu
from jax.experimental.pallas import tpu_sc as plsc

_D = 32
_NC = 2
_NS = 16
_NW = _NC * _NS
_LANES = 16
_BW = 128
_TCH = 2
_NBUF = 2
_REP = 1025


def _lookup(xt, wtab, n_t, n_b):
    chunks = n_t // _TCH
    mesh = plsc.VectorSubcoreMesh(core_axis_name="c", subcore_axis_name="s")

    @functools.partial(
        pl.kernel,
        mesh=mesh,
        compiler_params=pltpu.CompilerParams(needs_layout_passes=False),
        out_type=jax.ShapeDtypeStruct((n_t * _D, n_b), jnp.float32),
        scratch_types=[
            pltpu.VMEM((n_t, _BW), jnp.int32),
            pltpu.VMEM((_D * _D,), jnp.float32),
            pltpu.VMEM((_LANES * _REP,), jnp.float32),
        ] + [pltpu.VMEM((_TCH * _D, _BW), jnp.float32)] * _NBUF
          + [pltpu.SemaphoreType.DMA] * (_NBUF + 1),
    )
    def k(xt_hbm, wtab_hbm, out_hbm, xl_v, wtab_v, rep_v, *bufs_and_sems):
        rows_v = bufs_and_sems[:_NBUF]
        osem = bufs_and_sems[_NBUF:2 * _NBUF]
        xsem = bufs_and_sems[2 * _NBUF]
        wid = lax.axis_index("s") * _NC + lax.axis_index("c")
        b0 = wid * _BW
        iota = lax.iota(jnp.int32, _LANES)
        skew = iota * _REP

        pltpu.async_copy(xt_hbm.at[:, pl.ds(b0, _BW)], xl_v, xsem)
        pltpu.sync_copy(wtab_hbm, wtab_v)

        def rep_body(kk, carry):
            v = wtab_v[pl.ds(kk * _LANES, _LANES)]
            for l in range(_LANES):
                rep_v[pl.ds(l * _REP + kk * _LANES, _LANES)] = v
            return carry

        lax.fori_loop(0, (_D * _D) // _LANES, rep_body, 0)
        pltpu.make_async_copy(xt_hbm.at[:, pl.ds(0, _BW)], xl_v, xsem).wait()

        def pair_body(p, carry):
            for b in range(_NBUF):
                c = p * _NBUF + b
                t0 = c * _TCH
                @pl.when(p > 0)
                def _():
                    pltpu.make_async_copy(
                        rows_v[b], out_hbm.at[pl.ds(0, _TCH * _D),
                                              pl.ds(b0, _BW)],
                        osem[b]).wait()

                @plsc.parallel_loop(0, _TCH * (_BW // _LANES), 1, unroll=2)
                def group_body(i):
                    tl = i // (_BW // _LANES)
                    g = i % (_BW // _LANES)
                    cvec = xl_v[t0 + tl, pl.ds(g * _LANES, _LANES)]
                    coffs = jnp.minimum(jnp.maximum(cvec, 0), _D - 1)
                    bsvec = coffs + skew
                    for d in range(_D):
                        gth = plsc.load_gather(rep_v, [bsvec + d * _D])
                        rows_v[b][tl * _D + d, pl.ds(g * _LANES, _LANES)] = gth

                pltpu.async_copy(
                    rows_v[b],
                    out_hbm.at[pl.ds(t0 * _D, _TCH * _D), pl.ds(b0, _BW)],
                    osem[b])
            return carry

        lax.fori_loop(0, chunks // _NBUF, pair_body, 0)
        for b in range(_NBUF):
            pltpu.make_async_copy(
                rows_v[b], out_hbm.at[pl.ds(0, _TCH * _D), pl.ds(b0, _BW)],
                osem[b]).wait()

    return k(xt, wtab)


def kernel(x, W):
    n_b, n_t = x.shape
    wtab = W[:_D].T.reshape(-1)
    out2 = _lookup(x.T, wtab, n_t, n_b)
    out = out2.reshape(n_t, _D, n_b).transpose(2, 0, 1)
    return out
